# baseline (speedup 1.0000x reference)
import functools

import jax
import jax.numpy as jnp
from jax import lax
from jax.experimental import pallas as pl
from jax.experimental.pallas import tpu as pltpu
from jax.experimental.pallas import tpu_sc as plsc

B = 1024
L = 200
D = 32
BL = B * L

_NC = 2
_NS = 16
_NW = _NC * _NS

_PER_W = BL // _NW
_CHUNK = 128
_NCHUNK = _PER_W // _CHUNK


def _sc_gather3(t0, t1, t2, i0, i1, i2):
    mesh = plsc.VectorSubcoreMesh(core_axis_name="c", subcore_axis_name="s")

    @functools.partial(
        pl.kernel,
        out_type=[jax.ShapeDtypeStruct((BL, D), jnp.float32)] * 3,
        mesh=mesh,
        scratch_types=[
            pltpu.VMEM((_NCHUNK, _CHUNK), jnp.int32),
            pltpu.VMEM((_CHUNK, D), jnp.float32),
            pltpu.SemaphoreType.DMA,
        ],
        compiler_params=pltpu.CompilerParams(use_tc_tiling_on_sc=False),
    )
    def k(t0h, t1h, t2h, i0h, i1h, i2h, o0h, o1h, o2h, idx_v, rows_v, sem):
        wid = lax.axis_index("s") * _NC + lax.axis_index("c")
        tabs = (t0h, t1h, t2h)
        idxs = (i0h, i1h, i2h)
        outs = (o0h, o1h, o2h)
        for t in range(3):
            pltpu.sync_copy(idxs[t].at[wid], idx_v)
            tab = tabs[t]
            outh = outs[t]

            def body(j, carry):
                pltpu.async_copy(tab.at[idx_v.at[j]], rows_v, sem).wait()
                pltpu.sync_copy(
                    rows_v, outh.at[pl.ds(wid * _PER_W + j * _CHUNK, _CHUNK)]
                )
                return carry

            lax.fori_loop(0, _NCHUNK, body, 0)

    return k(t0, t1, t2, i0, i1, i2)


_TC_ROWS = 2048


_TC_BROWS = 8


def _tc_body(inter_ref, cont_ref, emb_ref, w_ref, b_ref, g_ref, beta_ref,
             mask_ref, gi_ref, out_i_ref, out_c_ref, mask_out_ref, gi_out_ref):
    iv = inter_ref[...]
    c = cont_ref[...]
    rows = emb_ref[...]
    r0 = rows[:, 0:1, :]
    r1 = rows[:, 1:2, :]
    r2 = rows[:, 2:3, :]
    out_i_ref[...] = jnp.where(iv == 0, r0, jnp.where(iv == 1, r1, r2))
    h = c * w_ref[...] + b_ref[...]
    mu = jnp.mean(h, axis=-1, keepdims=True)
    var = jnp.mean((h - mu) ** 2, axis=-1, keepdims=True)
    out_c_ref[...] = (h - mu) * lax.rsqrt(var + 1e-5) * g_ref[...] + beta_ref[...]
    mask_out_ref[...] = mask_ref[...]
    gi_out_ref[...] = gi_ref[...]


def _tc_dense(interaction, cont, emb_interaction, W_cont, b_cont, ln_gamma,
              ln_beta, mask, gi2d):
    grid = B // _TC_BROWS
    return pl.pallas_call(
        _tc_body,
        grid=(grid,),
        in_specs=[
            pl.BlockSpec((_TC_BROWS, L, 1), lambda i: (i, 0, 0)),
            pl.BlockSpec((_TC_BROWS, L, 1), lambda i: (i, 0, 0)),
            pl.BlockSpec((1, 3, D), lambda i: (0, 0, 0)),
            pl.BlockSpec((1, 1, D), lambda i: (0, 0, 0)),
            pl.BlockSpec((1, 1, D), lambda i: (0, 0, 0)),
            pl.BlockSpec((1, 1, D), lambda i: (0, 0, 0)),
            pl.BlockSpec((1, 1, D), lambda i: (0, 0, 0)),
            pl.BlockSpec((_TC_BROWS, L), lambda i: (i, 0)),
            pl.BlockSpec((1, B), lambda i: (0, 0)),
        ],
        out_specs=[
            pl.BlockSpec((_TC_BROWS, L, D), lambda i: (i, 0, 0)),
            pl.BlockSpec((_TC_BROWS, L, D), lambda i: (i, 0, 0)),
            pl.BlockSpec((_TC_BROWS, L), lambda i: (i, 0)),
            pl.BlockSpec((1, B), lambda i: (0, 0)),
        ],
        out_shape=[
            jax.ShapeDtypeStruct((B, L, D), jnp.float32),
            jax.ShapeDtypeStruct((B, L, D), jnp.float32),
            jax.ShapeDtypeStruct((B, L), jnp.float32),
            jax.ShapeDtypeStruct(gi2d.shape, gi2d.dtype),
        ],
    )(interaction, cont, emb_interaction, W_cont, b_cont, ln_gamma, ln_beta,
      mask, gi2d)


def kernel(cate_0, cate_1, cate_2, target, mask, interaction, cont_0,
           gather_index, emb_interaction, emb_cate_0, emb_cate_1, emb_cate_2,
           W_cont, b_cont, ln_gamma, ln_beta):
    e0, e1, e2 = _sc_gather3(
        emb_cate_0, emb_cate_1, emb_cate_2,
        cate_0.reshape(_NW, _NCHUNK, _CHUNK).astype(jnp.int32),
        cate_1.reshape(_NW, _NCHUNK, _CHUNK).astype(jnp.int32),
        cate_2.reshape(_NW, _NCHUNK, _CHUNK).astype(jnp.int32),
    )

    ei, ec, mask_o, gi_o = _tc_dense(
        interaction.reshape(B, L, 1).astype(jnp.int32),
        cont_0,
        emb_interaction.reshape(1, 3, D),
        W_cont.reshape(1, 1, D),
        b_cont.reshape(1, 1, D),
        ln_gamma.reshape(1, 1, D),
        ln_beta.reshape(1, 1, D),
        mask,
        gather_index.reshape(1, B),
    )

    return (
        e0.reshape(B, L, D),
        e1.reshape(B, L, D),
        e2.reshape(B, L, D),
        mask_o,
        ei,
        ec,
        gi_o.reshape(B),
    )


# speedup vs baseline: 2.2348x; 2.2348x over previous
import functools

import jax
import jax.numpy as jnp
from jax import lax
from jax.experimental import pallas as pl
from jax.experimental.pallas import tpu as pltpu
from jax.experimental.pallas import tpu_sc as plsc

B = 1024
L = 200
D = 32
BL = B * L

_NC = 2
_NS = 16
_NW = _NC * _NS

_PER_W = BL // _NW
_CHUNK = 128
_NCHUNK = _PER_W // _CHUNK


def _sc_gather(table, idx):
    mesh = plsc.VectorSubcoreMesh(core_axis_name="c", subcore_axis_name="s")

    @functools.partial(
        pl.kernel,
        out_type=jax.ShapeDtypeStruct((BL, D), jnp.float32),
        mesh=mesh,
        scratch_types=[
            pltpu.VMEM((_NCHUNK, _CHUNK), jnp.int32),
            pltpu.VMEM((_CHUNK, D), jnp.float32),
            pltpu.SemaphoreType.DMA,
        ],
        compiler_params=pltpu.CompilerParams(use_tc_tiling_on_sc=False),
    )
    def k(tab, idxh, outh, idx_v, rows_v, sem):
        wid = lax.axis_index("s") * _NC + lax.axis_index("c")
        pltpu.sync_copy(idxh.at[wid], idx_v)

        def body(j, carry):
            pltpu.async_copy(tab.at[idx_v.at[j]], rows_v, sem).wait()
            pltpu.sync_copy(
                rows_v, outh.at[pl.ds(wid * _PER_W + j * _CHUNK, _CHUNK)]
            )
            return carry

        lax.fori_loop(0, _NCHUNK, body, 0)

    return k(table, idx)


_LBLK = 8


def _tc_body(inter_ref, cont_ref, emb_ref, w_ref, b_ref, g_ref, beta_ref,
             mask_ref, gi_ref, out_i_ref, out_c_ref, mask_out_ref, gi_out_ref):
    emb = emb_ref[...]
    w = w_ref[...]
    bb = b_ref[...]
    g = g_ref[...]
    beta = beta_ref[...]
    wp = w - jnp.mean(w)
    bp = bb - jnp.mean(bb)
    A = jnp.mean(wp * wp)
    Bc = jnp.mean(wp * bp)
    C = jnp.mean(bp * bp)
    w1 = (wp * g).reshape(D, 1)
    w2 = (bp * g).reshape(D, 1)
    betac = beta.reshape(D, 1)
    r0 = emb[0:1, :].reshape(D, 1)
    r1 = emb[1:2, :].reshape(D, 1)
    r2 = emb[2:3, :].reshape(D, 1)
    for l in range(_LBLK):
        iv = inter_ref[l:l + 1, :]
        c = cont_ref[l:l + 1, :]
        sel = jnp.where(iv == 0, r0, jnp.where(iv == 1, r1, r2))
        out_i_ref[pl.ds(l * D, D), :] = sel
        s = lax.rsqrt((A * c + 2.0 * Bc) * c + C + 1e-5)
        out_c_ref[pl.ds(l * D, D), :] = w1 * (c * s) + w2 * s + betac
    mask_out_ref[...] = mask_ref[...]
    gi_out_ref[...] = gi_ref[...]


def _tc_dense(interT, contT, emb_interaction, W_cont, b_cont, ln_gamma,
              ln_beta, maskT, gi2d):
    grid = L // _LBLK
    return pl.pallas_call(
        _tc_body,
        grid=(grid,),
        in_specs=[
            pl.BlockSpec((_LBLK, B), lambda i: (i, 0)),
            pl.BlockSpec((_LBLK, B), lambda i: (i, 0)),
            pl.BlockSpec((3, D), lambda i: (0, 0)),
            pl.BlockSpec((1, D), lambda i: (0, 0)),
            pl.BlockSpec((1, D), lambda i: (0, 0)),
            pl.BlockSpec((1, D), lambda i: (0, 0)),
            pl.BlockSpec((1, D), lambda i: (0, 0)),
            pl.BlockSpec((_LBLK, B), lambda i: (i, 0)),
            pl.BlockSpec((1, B), lambda i: (0, 0)),
        ],
        out_specs=[
            pl.BlockSpec((_LBLK * D, B), lambda i: (i, 0)),
            pl.BlockSpec((_LBLK * D, B), lambda i: (i, 0)),
            pl.BlockSpec((_LBLK, B), lambda i: (i, 0)),
            pl.BlockSpec((1, B), lambda i: (0, 0)),
        ],
        out_shape=[
            jax.ShapeDtypeStruct((L * D, B), jnp.float32),
            jax.ShapeDtypeStruct((L * D, B), jnp.float32),
            jax.ShapeDtypeStruct((L, B), jnp.float32),
            jax.ShapeDtypeStruct(gi2d.shape, gi2d.dtype),
        ],
    )(interT, contT, emb_interaction, W_cont, b_cont, ln_gamma, ln_beta,
      maskT, gi2d)


def kernel(cate_0, cate_1, cate_2, target, mask, interaction, cont_0,
           gather_index, emb_interaction, emb_cate_0, emb_cate_1, emb_cate_2,
           W_cont, b_cont, ln_gamma, ln_beta):
    e0 = _sc_gather(emb_cate_0,
                    cate_0.reshape(_NW, _NCHUNK, _CHUNK).astype(jnp.int32))
    e1 = _sc_gather(emb_cate_1,
                    cate_1.reshape(_NW, _NCHUNK, _CHUNK).astype(jnp.int32))
    e2 = _sc_gather(emb_cate_2,
                    cate_2.reshape(_NW, _NCHUNK, _CHUNK).astype(jnp.int32))

    ei_t, ec_t, mask_t, gi_o = _tc_dense(
        interaction.T.astype(jnp.int32),
        cont_0.reshape(B, L).T,
        emb_interaction,
        W_cont.reshape(1, D),
        b_cont.reshape(1, D),
        ln_gamma.reshape(1, D),
        ln_beta.reshape(1, D),
        mask.T,
        gather_index.reshape(1, B),
    )
    ei = ei_t.reshape(L, D, B).transpose(2, 0, 1)
    ec = ec_t.reshape(L, D, B).transpose(2, 0, 1)

    return (
        e0.reshape(B, L, D),
        e1.reshape(B, L, D),
        e2.reshape(B, L, D),
        mask_t.T,
        ei,
        ec,
        gi_o.reshape(B),
    )


# speedup vs baseline: 2.8122x; 1.2584x over previous
import functools

import jax
import jax.numpy as jnp
from jax import lax
from jax.experimental import pallas as pl
from jax.experimental.pallas import tpu as pltpu
from jax.experimental.pallas import tpu_sc as plsc

B = 1024
L = 200
D = 32
BL = B * L

_NC = 2
_NS = 16
_NW = _NC * _NS

_PER_W = BL // _NW
_CHUNK = 128
_NCHUNK = _PER_W // _CHUNK


def _sc_gather(table, idx):
    mesh = plsc.VectorSubcoreMesh(core_axis_name="c", subcore_axis_name="s")

    @functools.partial(
        pl.kernel,
        out_type=jax.ShapeDtypeStruct((BL, D), jnp.float32),
        mesh=mesh,
        scratch_types=[
            pltpu.VMEM((_NCHUNK, _CHUNK), jnp.int32),
            pltpu.VMEM((_CHUNK, D), jnp.float32),
            pltpu.SemaphoreType.DMA,
        ],
        compiler_params=pltpu.CompilerParams(use_tc_tiling_on_sc=False),
    )
    def k(tab, idxh, outh, idx_v, rows_v, sem):
        wid = lax.axis_index("s") * _NC + lax.axis_index("c")
        pltpu.sync_copy(idxh.at[wid], idx_v)

        def body(j, carry):
            pltpu.async_copy(tab.at[idx_v.at[j]], rows_v, sem).wait()
            pltpu.sync_copy(
                rows_v, outh.at[pl.ds(wid * _PER_W + j * _CHUNK, _CHUNK)]
            )
            return carry

        lax.fori_loop(0, _NCHUNK, body, 0)

    return k(table, idx)


_TB = 1024
_TC = 640


def _tr_body(x_ref, o_ref):
    o_ref[...] = x_ref[...].T


def _tc_transpose(x):
    xv = x.reshape(B, L * D)
    return pl.pallas_call(
        _tr_body,
        grid=(B // _TB, (L * D) // _TC),
        in_specs=[pl.BlockSpec((_TB, _TC), lambda i, j: (i, j))],
        out_specs=pl.BlockSpec((_TC, _TB), lambda i, j: (j, i)),
        out_shape=jax.ShapeDtypeStruct((L * D, B), jnp.float32),
    )(xv)


_LBLK = 8


def _tc_body(inter_ref, cont_ref, emb_ref, w_ref, b_ref, g_ref, beta_ref,
             mask_ref, gi_ref, out_i_ref, out_c_ref, mask_out_ref, gi_out_ref):
    emb = emb_ref[...]
    w = w_ref[...]
    bb = b_ref[...]
    g = g_ref[...]
    beta = beta_ref[...]
    wp = w - jnp.mean(w)
    bp = bb - jnp.mean(bb)
    A = jnp.mean(wp * wp)
    Bc = jnp.mean(wp * bp)
    C = jnp.mean(bp * bp)
    w1 = (wp * g).reshape(D, 1)
    w2 = (bp * g).reshape(D, 1)
    betac = beta.reshape(D, 1)
    r0 = emb[0:1, :].reshape(D, 1)
    r1 = emb[1:2, :].reshape(D, 1)
    r2 = emb[2:3, :].reshape(D, 1)
    for l in range(_LBLK):
        iv = inter_ref[l:l + 1, :]
        c = cont_ref[l:l + 1, :]
        sel = jnp.where(iv == 0, r0, jnp.where(iv == 1, r1, r2))
        out_i_ref[pl.ds(l * D, D), :] = sel
        s = lax.rsqrt((A * c + 2.0 * Bc) * c + C + 1e-5)
        out_c_ref[pl.ds(l * D, D), :] = w1 * (c * s) + w2 * s + betac
    mask_out_ref[...] = mask_ref[...]
    gi_out_ref[...] = gi_ref[...]


def _tc_dense(interT, contT, emb_interaction, W_cont, b_cont, ln_gamma,
              ln_beta, maskT, gi2d):
    grid = L // _LBLK
    return pl.pallas_call(
        _tc_body,
        grid=(grid,),
        in_specs=[
            pl.BlockSpec((_LBLK, B), lambda i: (i, 0)),
            pl.BlockSpec((_LBLK, B), lambda i: (i, 0)),
            pl.BlockSpec((3, D), lambda i: (0, 0)),
            pl.BlockSpec((1, D), lambda i: (0, 0)),
            pl.BlockSpec((1, D), lambda i: (0, 0)),
            pl.BlockSpec((1, D), lambda i: (0, 0)),
            pl.BlockSpec((1, D), lambda i: (0, 0)),
            pl.BlockSpec((_LBLK, B), lambda i: (i, 0)),
            pl.BlockSpec((1, B), lambda i: (0, 0)),
        ],
        out_specs=[
            pl.BlockSpec((_LBLK * D, B), lambda i: (i, 0)),
            pl.BlockSpec((_LBLK * D, B), lambda i: (i, 0)),
            pl.BlockSpec((_LBLK, B), lambda i: (i, 0)),
            pl.BlockSpec((1, B), lambda i: (0, 0)),
        ],
        out_shape=[
            jax.ShapeDtypeStruct((L * D, B), jnp.float32),
            jax.ShapeDtypeStruct((L * D, B), jnp.float32),
            jax.ShapeDtypeStruct((L, B), jnp.float32),
            jax.ShapeDtypeStruct(gi2d.shape, gi2d.dtype),
        ],
    )(interT, contT, emb_interaction, W_cont, b_cont, ln_gamma, ln_beta,
      maskT, gi2d)


def kernel(cate_0, cate_1, cate_2, target, mask, interaction, cont_0,
           gather_index, emb_interaction, emb_cate_0, emb_cate_1, emb_cate_2,
           W_cont, b_cont, ln_gamma, ln_beta):
    e0 = _sc_gather(emb_cate_0,
                    cate_0.reshape(_NW, _NCHUNK, _CHUNK).astype(jnp.int32))
    e1 = _sc_gather(emb_cate_1,
                    cate_1.reshape(_NW, _NCHUNK, _CHUNK).astype(jnp.int32))
    e2 = _sc_gather(emb_cate_2,
                    cate_2.reshape(_NW, _NCHUNK, _CHUNK).astype(jnp.int32))

    ei_t, ec_t, mask_t, gi_o = _tc_dense(
        interaction.T.astype(jnp.int32),
        cont_0.reshape(B, L).T,
        emb_interaction,
        W_cont.reshape(1, D),
        b_cont.reshape(1, D),
        ln_gamma.reshape(1, D),
        ln_beta.reshape(1, D),
        mask.T,
        gather_index.reshape(1, B),
    )
    ei = ei_t.reshape(L, D, B).transpose(2, 0, 1)
    ec = ec_t.reshape(L, D, B).transpose(2, 0, 1)

    e0t = _tc_transpose(e0).reshape(L, D, B).transpose(2, 0, 1)
    e1t = _tc_transpose(e1).reshape(L, D, B).transpose(2, 0, 1)
    e2t = _tc_transpose(e2).reshape(L, D, B).transpose(2, 0, 1)

    return (
        e0t,
        e1t,
        e2t,
        mask_t.T,
        ei,
        ec,
        gi_o.reshape(B),
    )
